# baseline (speedup 1.0000x reference)
import functools

import jax
import jax.numpy as jnp
from jax import lax
from jax.experimental import pallas as pl
from jax.experimental.pallas import tpu as pltpu
from jax.experimental.pallas import tpu_sc as plsc

_B, _T = 16384, 4096
_BLOCK_ROWS = 512
_NC, _NS = 2, 16
_NW = _NC * _NS
_ROWS_PER_W = _B // _NW
_SLAB = 128
_L = 16
_UNROLL = 8


@functools.partial(
    pl.kernel,
    out_type=jax.ShapeDtypeStruct((_B, _SLAB), jnp.float32),
    mesh=plsc.VectorSubcoreMesh(core_axis_name="c", subcore_axis_name="s"),
    scratch_types=[
        pltpu.VMEM((_ROWS_PER_W, _SLAB), jnp.float32),
    ],
)
def _sc_swap(tokens_ref, out_ref, blk):
    wid = lax.axis_index("s") * _NC + lax.axis_index("c")
    base = wid * _ROWS_PER_W
    pltpu.sync_copy(tokens_ref.at[pl.ds(base, _ROWS_PER_W), pl.ds(0, _SLAB)], blk)
    lane = lax.iota(jnp.int32, _L)
    perm = jnp.where(lane == 0, 1, jnp.where(lane == 1, 0, lane))
    dnums = lax.GatherDimensionNumbers(
        offset_dims=(), collapsed_slice_dims=(0,), start_index_map=(0,))

    def body(i, carry):
        for u in range(_UNROLL):
            r = i * _UNROLL + u
            v = blk[r, pl.ds(0, _L)]
            swapped = lax.gather(
                v, perm[:, None], dnums, (1,),
                mode=lax.GatherScatterMode.PROMISE_IN_BOUNDS)
            blk[r, pl.ds(0, _L)] = swapped
        return carry

    lax.fori_loop(0, _ROWS_PER_W // _UNROLL, body, 0)
    pltpu.sync_copy(blk, out_ref.at[pl.ds(base, _ROWS_PER_W), pl.ds(0, _SLAB)])


def _copy_body(x_ref, o_ref):
    o_ref[...] = x_ref[...]


def _tc_copy_big(tokens):
    return pl.pallas_call(
        _copy_body,
        grid=(_B // _BLOCK_ROWS, _T // _SLAB - 1),
        in_specs=[pl.BlockSpec((_BLOCK_ROWS, _SLAB), lambda i, j: (i, j + 1))],
        out_specs=pl.BlockSpec((_BLOCK_ROWS, _SLAB), lambda i, j: (i, j + 1)),
        out_shape=jax.ShapeDtypeStruct((_B, _T), tokens.dtype),
    )(tokens)


def _insert_body(s_ref, y_ref, o_ref):
    del y_ref
    o_ref[...] = s_ref[...]


def _tc_insert(sc_slab, y0):
    return pl.pallas_call(
        _insert_body,
        grid=(_B // _BLOCK_ROWS,),
        in_specs=[
            pl.BlockSpec((_BLOCK_ROWS, _SLAB), lambda i: (i, 0)),
            pl.BlockSpec(memory_space=pl.ANY),
        ],
        out_specs=pl.BlockSpec((_BLOCK_ROWS, _SLAB), lambda i: (i, 0)),
        out_shape=jax.ShapeDtypeStruct((_B, _T), y0.dtype),
        input_output_aliases={1: 0},
    )(sc_slab, y0)


def kernel(tokens):
    sc_slab = _sc_swap(tokens)
    y0 = _tc_copy_big(tokens)
    return _tc_insert(sc_slab, y0)


# speedup vs baseline: 3.4839x; 3.4839x over previous
import functools

import jax
import jax.numpy as jnp
from jax import lax
from jax.experimental import pallas as pl
from jax.experimental.pallas import tpu as pltpu
from jax.experimental.pallas import tpu_sc as plsc

_B, _T = 16384, 4096
_BLOCK_ROWS = 512
_NC, _NS = 2, 16
_NW = _NC * _NS
_ROWS_PER_W = _B // _NW
_SLAB = 128
_L = 16
_UNROLL = 8


@functools.partial(
    pl.kernel,
    mesh=plsc.VectorSubcoreMesh(core_axis_name="c", subcore_axis_name="s"),
    scratch_types=[
        pltpu.VMEM((_ROWS_PER_W, _SLAB), jnp.float32),
    ],
)
def _sc_swap(y_ref, blk):
    wid = lax.axis_index("s") * _NC + lax.axis_index("c")
    base = wid * _ROWS_PER_W
    pltpu.sync_copy(y_ref.at[pl.ds(base, _ROWS_PER_W), pl.ds(0, _SLAB)], blk)
    lane = lax.iota(jnp.int32, _L)
    perm = jnp.where(lane == 0, 1, jnp.where(lane == 1, 0, lane))
    dnums = lax.GatherDimensionNumbers(
        offset_dims=(), collapsed_slice_dims=(0,), start_index_map=(0,))

    def body(i, carry):
        for u in range(_UNROLL):
            r = i * _UNROLL + u
            v = blk[r, pl.ds(0, _L)]
            swapped = lax.gather(
                v, perm[:, None], dnums, (1,),
                mode=lax.GatherScatterMode.PROMISE_IN_BOUNDS)
            blk[r, pl.ds(0, _L)] = swapped
        return carry

    lax.fori_loop(0, _ROWS_PER_W // _UNROLL, body, 0)
    pltpu.sync_copy(blk, y_ref.at[pl.ds(base, _ROWS_PER_W), pl.ds(0, _SLAB)])


def _copy_body(x_ref, o_ref):
    o_ref[...] = x_ref[...]


def _tc_copy(tokens):
    return pl.pallas_call(
        _copy_body,
        grid=(_B // _BLOCK_ROWS,),
        in_specs=[pl.BlockSpec((_BLOCK_ROWS, _T), lambda i: (i, 0))],
        out_specs=pl.BlockSpec((_BLOCK_ROWS, _T), lambda i: (i, 0)),
        out_shape=jax.ShapeDtypeStruct((_B, _T), tokens.dtype),
    )(tokens)


def kernel(tokens):
    y_ref = jax.new_ref(_tc_copy(tokens))
    _sc_swap(y_ref)
    return jax.freeze(y_ref)
